# baseline (speedup 1.0000x reference)
import functools

import jax
import jax.numpy as jnp
from jax import lax
from jax.experimental import pallas as pl
from jax.experimental.pallas import tpu as pltpu
from jax.experimental.pallas import tpu_sc as plsc

N = 10000
E = 320000
B = 8
L = N // B
D = 128
T = 4
STEPS = 6
CD = 2 * D

NPAD = 10240
NC = 2
NS = 16
DH = D // NC
CHUNK = 256
NBUF = 2
NCH = (-(-E // (NS * CHUNK)) + NBUF - 1) // NBUF * NBUF
EPW = NCH * CHUNK


def _trans_body(hx_ref, w_ref, b_ref, out_ref):
    x = hx_ref[...]
    w = w_ref[0]
    y = lax.dot_general(x, w, (((1,), (1,)), ((), ())),
                        preferred_element_type=jnp.float32)
    t = pl.program_id(0)
    y = y + b_ref[t][None, :]
    out_ref[0, 0] = y[:, :DH]
    out_ref[1, 0] = y[:, DH:]


def _trans(hx, W_msg, b_msg):
    BN = 2048
    return pl.pallas_call(
        _trans_body,
        grid=(T, NPAD // BN),
        in_specs=[
            pl.BlockSpec((BN, D), lambda t, i: (i, 0)),
            pl.BlockSpec((1, D, D), lambda t, i: (t, 0, 0)),
            pl.BlockSpec((T, D), lambda t, i: (0, 0)),
        ],
        out_specs=pl.BlockSpec((NC, 1, BN, DH), lambda t, i: (0, t, i, 0)),
        out_shape=jax.ShapeDtypeStruct((NC, T, NPAD, DH), jnp.float32),
    )(hx, W_msg, b_msg)


def _sc_gather_scatter(trans_flat, gidx_all, dst_all, zinit):
    mesh = plsc.VectorSubcoreMesh(core_axis_name="c", subcore_axis_name="s")

    @functools.partial(
        pl.kernel,
        mesh=mesh,
        compiler_params=pltpu.CompilerParams(use_tc_tiling_on_sc=False),
        out_type=jax.ShapeDtypeStruct((NC, NPAD, DH), jnp.float32),
        scratch_types=[
            pltpu.VMEM((NCH, CHUNK), jnp.int32),
            pltpu.VMEM((NCH, CHUNK), jnp.int32),
            pltpu.VMEM_SHARED((NPAD, DH), jnp.float32),
        ] + [pltpu.VMEM((CHUNK, DH), jnp.float32) for _ in range(NBUF)]
          + [pltpu.SemaphoreType.DMA for _ in range(2 * NBUF)],
    )
    def run(trans_hbm, gidx_hbm, dst_hbm, zinit_hbm, out_hbm,
            gv, dv, acc, *bufs_sems):
        rbufs = bufs_sems[:NBUF]
        gsems = bufs_sems[NBUF:2 * NBUF]
        ssems = bufs_sems[2 * NBUF:]
        c = lax.axis_index("c")
        s = lax.axis_index("s")
        pltpu.sync_copy(gidx_hbm.at[c, s], gv)
        pltpu.sync_copy(dst_hbm.at[s], dv)
        rpt = NPAD // NS
        pltpu.sync_copy(zinit_hbm.at[pl.ds(s * rpt, rpt)],
                        acc.at[pl.ds(s * rpt, rpt)])
        plsc.subcore_barrier()

        def g_desc(b, j):
            return pltpu.make_async_copy(trans_hbm.at[gv.at[j]], rbufs[b],
                                         gsems[b])

        def s_desc(b, j):
            return pltpu.make_async_copy(rbufs[b], acc.at[dv.at[j]], ssems[b])

        for b in range(NBUF):
            g_desc(b, b).start()

        def body(jj, _):
            j0 = jj * NBUF
            for b in range(NBUF):
                g_desc(b, j0 + b).wait()
                s_desc(b, j0 + b).start(add=True)
            for b in range(NBUF):
                s_desc(b, j0 + b).wait()

                @pl.when(j0 + b + NBUF < NCH)
                def _():
                    g_desc(b, j0 + b + NBUF).start()
            return 0

        lax.fori_loop(0, NCH // NBUF, body, 0)
        plsc.subcore_barrier()
        pltpu.sync_copy(acc.at[pl.ds(s * rpt, rpt)],
                        out_hbm.at[c, pl.ds(s * rpt, rpt)])

    return run(trans_flat, gidx_all, dst_all, zinit)


def _gru_trans_body(p_ref, hx_ref, wih_ref, bih_ref, whh_ref, bhh_ref,
                    wmsg_ref, bmsg_ref, out_ref, tr_ref):
    a = jnp.concatenate([p_ref[0], p_ref[1]], axis=1)
    x = hx_ref[...]
    gi = lax.dot_general(a, wih_ref[...], (((1,), (1,)), ((), ())),
                         preferred_element_type=jnp.float32) + bih_ref[...][None, :]
    gh = lax.dot_general(x, whh_ref[...], (((1,), (1,)), ((), ())),
                         preferred_element_type=jnp.float32) + bhh_ref[...][None, :]
    r = jax.nn.sigmoid(gi[:, :D] + gh[:, :D])
    z = jax.nn.sigmoid(gi[:, D:2 * D] + gh[:, D:2 * D])
    n = jnp.tanh(gi[:, 2 * D:] + r * gh[:, 2 * D:])
    hx_new = (1.0 - z) * n + z * x
    out_ref[...] = hx_new
    for t in range(T):
        y = lax.dot_general(hx_new, wmsg_ref[t], (((1,), (1,)), ((), ())),
                            preferred_element_type=jnp.float32)
        y = y + bmsg_ref[t][None, :]
        tr_ref[0, t] = y[:, :DH]
        tr_ref[1, t] = y[:, DH:]


def _gru_trans(partials, hx, W_ih, b_ih, W_hh, b_hh, W_msg, b_msg):
    BG = 2048
    return pl.pallas_call(
        _gru_trans_body,
        grid=(NPAD // BG,),
        in_specs=[
            pl.BlockSpec((NC, BG, DH), lambda i: (0, i, 0)),
            pl.BlockSpec((BG, D), lambda i: (i, 0)),
            pl.BlockSpec((3 * D, D), lambda i: (0, 0)),
            pl.BlockSpec((3 * D,), lambda i: (0,)),
            pl.BlockSpec((3 * D, D), lambda i: (0, 0)),
            pl.BlockSpec((3 * D,), lambda i: (0,)),
            pl.BlockSpec((T, D, D), lambda i: (0, 0, 0)),
            pl.BlockSpec((T, D), lambda i: (0, 0)),
        ],
        out_specs=[
            pl.BlockSpec((BG, D), lambda i: (i, 0)),
            pl.BlockSpec((NC, T, BG, DH), lambda i: (0, 0, i, 0)),
        ],
        out_shape=[
            jax.ShapeDtypeStruct((NPAD, D), jnp.float32),
            jax.ShapeDtypeStruct((NC, T, NPAD, DH), jnp.float32),
        ],
    )(partials, hx, W_ih, b_ih, W_hh, b_hh, W_msg, b_msg)


L1 = L - 2
P1 = (L1 - 3) // 2 + 1
P2 = (P1 - 2) // 2 + 1


def _conv3(x, w_ref, b, n_out):
    acc = None
    for k in range(3):
        xk = x[k:k + L1]
        yk = lax.dot_general(xk, w_ref[k], (((1,), (1,)), ((), ())),
                             preferred_element_type=jnp.float32)
        acc = yk if acc is None else acc + yk
    return jnp.maximum(acc + b[None, :], 0.0)


def _pool3(m3):
    c = m3.shape[1]
    r = m3.reshape(L1 // 2, 2, c)
    ev = r[:, 0, :]
    od = r[:, 1, :]
    pair = jnp.maximum(ev, od)
    return jnp.maximum(pair[0:P1], ev[1:P1 + 1])


def _conv1(x, w_ref, b):
    y = lax.dot_general(x, w_ref[0], (((1,), (1,)), ((), ())),
                        preferred_element_type=jnp.float32)
    return jnp.maximum(y + b[None, :], 0.0)


def _pool2(c2):
    c = c2.shape[1]
    r = c2[0:2 * P2].reshape(P2, 2, c)
    return jnp.maximum(r[:, 0, :], r[:, 1, :])


def _readout_body(hf_ref, ft_ref, wc1_ref, bc1_ref, wc2_ref, bc2_ref,
                  wcc1_ref, bcc1_ref, wcc2_ref, bcc2_ref,
                  wy_ref, by_ref, wz_ref, bz_ref, out_ref):
    hi = hf_ref[0]
    ft = ft_ref[0]
    y2 = _pool2(_conv1(_pool3(_conv3(hi, wc1_ref, bc1_ref[...], D)),
                       wc2_ref, bc2_ref[...]))
    ci = jnp.concatenate([hi, ft], axis=1)
    z2 = _pool2(_conv1(_pool3(_conv3(ci, wcc1_ref, bcc1_ref[...], CD)),
                       wcc2_ref, bcc2_ref[...]))
    M = lax.dot_general(y2, z2, (((0,), (0,)), ((), ())),
                        preferred_element_type=jnp.float32)
    W = lax.dot_general(wy_ref[...], wz_ref[...], (((0,), (0,)), ((), ())),
                        preferred_element_type=jnp.float32)
    by = by_ref[0]
    bz = bz_ref[0]
    val = (jnp.sum(W * M)
           + by * jnp.sum(z2 * wz_ref[...])
           + bz * jnp.sum(y2 * wy_ref[...])
           + float(P2) * by * bz) / float(P2)
    out_ref[pl.program_id(0), :] = jnp.full((128,), jax.nn.sigmoid(val),
                                            jnp.float32)


def _readout(hfin, feat, w_c1, b_c1, w_c2, b_c2, w_cc1, b_cc1, w_cc2, b_cc2,
             W_y, b_y, W_z, b_z):
    full = lambda *shape: pl.BlockSpec(shape, lambda b: (0,) * len(shape))
    return pl.pallas_call(
        _readout_body,
        grid=(B,),
        in_specs=[
            pl.BlockSpec((1, L, D), lambda b: (b, 0, 0)),
            pl.BlockSpec((1, L, D), lambda b: (b, 0, 0)),
            full(3, D, D), full(D,), full(1, D, D), full(D,),
            full(3, CD, CD), full(CD,), full(1, CD, CD), full(CD,),
            full(1, D),
            pl.BlockSpec(memory_space=pltpu.SMEM),
            full(1, CD),
            pl.BlockSpec(memory_space=pltpu.SMEM),
        ],
        out_specs=pl.BlockSpec((B, 128), lambda b: (0, 0)),
        out_shape=jax.ShapeDtypeStruct((B, 128), jnp.float32),
    )(hfin, feat, w_c1, b_c1, w_c2, b_c2, w_cc1, b_cc1, w_cc2, b_cc2,
      W_y, b_y, W_z, b_z)


def kernel(h, edge_index, etype, W_msg, b_msg, W_ih, b_ih, W_hh, b_hh,
           w_c1, b_c1, w_c2, b_c2, w_cc1, b_cc1, w_cc2, b_cc2,
           W_y, b_y, W_z, b_z):
    src = edge_index[0]
    dst = edge_index[1]
    gidx = etype * NPAD + src
    npad_e = NS * EPW - E
    pad_ids = jnp.arange(npad_e, dtype=jnp.int32)
    gidx_t = jnp.concatenate(
        [gidx, (pad_ids * 97) % (T * NPAD)]).reshape(NS, NCH, CHUNK)
    gidx_all = jnp.stack([gidx_t + c * (T * NPAD) for c in range(NC)])
    dst_all = jnp.concatenate(
        [dst, N + pad_ids % (NPAD - N)]).reshape(NS, NCH, CHUNK)
    zinit = jnp.zeros((NPAD, DH), jnp.float32)
    hx = jnp.pad(h, ((0, NPAD - N), (0, 0)))

    trans = _trans(hx, W_msg, b_msg)
    for _ in range(STEPS):
        partials = _sc_gather_scatter(trans.reshape(NC * T * NPAD, DH),
                                      gidx_all, dst_all, zinit)
        hx, trans = _gru_trans(partials, hx, W_ih, b_ih, W_hh, b_hh,
                               W_msg, b_msg)

    hfin = hx[:N].reshape(B, L, D)
    feat = h.reshape(B, L, D)
    out = _readout(hfin, feat,
                   jnp.transpose(w_c1, (2, 0, 1)), b_c1,
                   jnp.transpose(w_c2, (2, 0, 1)), b_c2,
                   jnp.transpose(w_cc1, (2, 0, 1)), b_cc1,
                   jnp.transpose(w_cc2, (2, 0, 1)), b_cc2,
                   W_y, b_y, W_z, b_z)
    return out[:, 0]


# speedup vs baseline: 1.1837x; 1.1837x over previous
import functools

import jax
import jax.numpy as jnp
from jax import lax
from jax.experimental import pallas as pl
from jax.experimental.pallas import tpu as pltpu
from jax.experimental.pallas import tpu_sc as plsc

N = 10000
E = 320000
B = 8
L = N // B
D = 128
T = 4
STEPS = 6
CD = 2 * D

NPAD = 10240
NC = 2
NS = 16
DH = D // NC
CHUNK = 128
NBUF = 5
NCH = (-(-E // (NS * CHUNK)) + NBUF - 1) // NBUF * NBUF
EPW = NCH * CHUNK


def _trans_body(hx_ref, w_ref, b_ref, out_ref):
    x = hx_ref[...]
    w = w_ref[0]
    y = lax.dot_general(x, w, (((1,), (1,)), ((), ())),
                        preferred_element_type=jnp.float32)
    t = pl.program_id(0)
    y = y + b_ref[t][None, :]
    out_ref[0, 0] = y[:, :DH]
    out_ref[1, 0] = y[:, DH:]


def _trans(hx, W_msg, b_msg):
    BN = 2048
    return pl.pallas_call(
        _trans_body,
        grid=(T, NPAD // BN),
        in_specs=[
            pl.BlockSpec((BN, D), lambda t, i: (i, 0)),
            pl.BlockSpec((1, D, D), lambda t, i: (t, 0, 0)),
            pl.BlockSpec((T, D), lambda t, i: (0, 0)),
        ],
        out_specs=pl.BlockSpec((NC, 1, BN, DH), lambda t, i: (0, t, i, 0)),
        out_shape=jax.ShapeDtypeStruct((NC, T, NPAD, DH), jnp.float32),
    )(hx, W_msg, b_msg)


def _sc_gather_scatter(trans_flat, gidx_all, dst_all, zinit):
    mesh = plsc.VectorSubcoreMesh(core_axis_name="c", subcore_axis_name="s")

    @functools.partial(
        pl.kernel,
        mesh=mesh,
        compiler_params=pltpu.CompilerParams(use_tc_tiling_on_sc=False),
        out_type=jax.ShapeDtypeStruct((NC, NPAD, DH), jnp.float32),
        scratch_types=[
            pltpu.VMEM((NCH, CHUNK), jnp.int32),
            pltpu.VMEM((NCH, CHUNK), jnp.int32),
            pltpu.VMEM_SHARED((NPAD, DH), jnp.float32),
        ] + [pltpu.VMEM((CHUNK, DH), jnp.float32) for _ in range(NBUF)]
          + [pltpu.SemaphoreType.DMA for _ in range(2 * NBUF)],
    )
    def run(trans_hbm, gidx_hbm, dst_hbm, zinit_hbm, out_hbm,
            gv, dv, acc, *bufs_sems):
        rbufs = bufs_sems[:NBUF]
        gsems = bufs_sems[NBUF:2 * NBUF]
        ssems = bufs_sems[2 * NBUF:]
        c = lax.axis_index("c")
        s = lax.axis_index("s")
        pltpu.sync_copy(gidx_hbm.at[c, s], gv)
        pltpu.sync_copy(dst_hbm.at[s], dv)
        rpt = NPAD // NS
        pltpu.sync_copy(zinit_hbm.at[pl.ds(s * rpt, rpt)],
                        acc.at[pl.ds(s * rpt, rpt)])
        plsc.subcore_barrier()

        def g_desc(b, j):
            return pltpu.make_async_copy(trans_hbm.at[gv.at[j]], rbufs[b],
                                         gsems[b])

        def s_desc(b, j):
            return pltpu.make_async_copy(rbufs[b], acc.at[dv.at[j]], ssems[b])

        for b in range(NBUF):
            g_desc(b, b).start()

        def body(jj, _):
            j0 = jj * NBUF
            for b in range(NBUF):
                g_desc(b, j0 + b).wait()
                s_desc(b, j0 + b).start(add=True)
            for b in range(NBUF):
                s_desc(b, j0 + b).wait()

                @pl.when(j0 + b + NBUF < NCH)
                def _():
                    g_desc(b, j0 + b + NBUF).start()
            return 0

        lax.fori_loop(0, NCH // NBUF, body, 0)
        plsc.subcore_barrier()
        pltpu.sync_copy(acc.at[pl.ds(s * rpt, rpt)],
                        out_hbm.at[c, pl.ds(s * rpt, rpt)])

    return run(trans_flat, gidx_all, dst_all, zinit)


def _gru_trans_body(p_ref, hx_ref, wih_ref, bih_ref, whh_ref, bhh_ref,
                    wmsg_ref, bmsg_ref, out_ref, tr_ref):
    a = jnp.concatenate([p_ref[0], p_ref[1]], axis=1)
    x = hx_ref[...]
    gi = lax.dot_general(a, wih_ref[...], (((1,), (1,)), ((), ())),
                         preferred_element_type=jnp.float32) + bih_ref[...][None, :]
    gh = lax.dot_general(x, whh_ref[...], (((1,), (1,)), ((), ())),
                         preferred_element_type=jnp.float32) + bhh_ref[...][None, :]
    r = jax.nn.sigmoid(gi[:, :D] + gh[:, :D])
    z = jax.nn.sigmoid(gi[:, D:2 * D] + gh[:, D:2 * D])
    n = jnp.tanh(gi[:, 2 * D:] + r * gh[:, 2 * D:])
    hx_new = (1.0 - z) * n + z * x
    out_ref[...] = hx_new
    for t in range(T):
        y = lax.dot_general(hx_new, wmsg_ref[t], (((1,), (1,)), ((), ())),
                            preferred_element_type=jnp.float32)
        y = y + bmsg_ref[t][None, :]
        tr_ref[0, t] = y[:, :DH]
        tr_ref[1, t] = y[:, DH:]


def _gru_trans(partials, hx, W_ih, b_ih, W_hh, b_hh, W_msg, b_msg):
    BG = 2048
    return pl.pallas_call(
        _gru_trans_body,
        grid=(NPAD // BG,),
        in_specs=[
            pl.BlockSpec((NC, BG, DH), lambda i: (0, i, 0)),
            pl.BlockSpec((BG, D), lambda i: (i, 0)),
            pl.BlockSpec((3 * D, D), lambda i: (0, 0)),
            pl.BlockSpec((3 * D,), lambda i: (0,)),
            pl.BlockSpec((3 * D, D), lambda i: (0, 0)),
            pl.BlockSpec((3 * D,), lambda i: (0,)),
            pl.BlockSpec((T, D, D), lambda i: (0, 0, 0)),
            pl.BlockSpec((T, D), lambda i: (0, 0)),
        ],
        out_specs=[
            pl.BlockSpec((BG, D), lambda i: (i, 0)),
            pl.BlockSpec((NC, T, BG, DH), lambda i: (0, 0, i, 0)),
        ],
        out_shape=[
            jax.ShapeDtypeStruct((NPAD, D), jnp.float32),
            jax.ShapeDtypeStruct((NC, T, NPAD, DH), jnp.float32),
        ],
    )(partials, hx, W_ih, b_ih, W_hh, b_hh, W_msg, b_msg)


L1 = L - 2
P1 = (L1 - 3) // 2 + 1
P2 = (P1 - 2) // 2 + 1


def _conv3(x, w_ref, b, n_out):
    acc = None
    for k in range(3):
        xk = x[k:k + L1]
        yk = lax.dot_general(xk, w_ref[k], (((1,), (1,)), ((), ())),
                             preferred_element_type=jnp.float32)
        acc = yk if acc is None else acc + yk
    return jnp.maximum(acc + b[None, :], 0.0)


def _pool3(m3):
    c = m3.shape[1]
    r = m3.reshape(L1 // 2, 2, c)
    ev = r[:, 0, :]
    od = r[:, 1, :]
    pair = jnp.maximum(ev, od)
    return jnp.maximum(pair[0:P1], ev[1:P1 + 1])


def _conv1(x, w_ref, b):
    y = lax.dot_general(x, w_ref[0], (((1,), (1,)), ((), ())),
                        preferred_element_type=jnp.float32)
    return jnp.maximum(y + b[None, :], 0.0)


def _pool2(c2):
    c = c2.shape[1]
    r = c2[0:2 * P2].reshape(P2, 2, c)
    return jnp.maximum(r[:, 0, :], r[:, 1, :])


def _readout_body(hf_ref, ft_ref, wc1_ref, bc1_ref, wc2_ref, bc2_ref,
                  wcc1_ref, bcc1_ref, wcc2_ref, bcc2_ref,
                  wy_ref, by_ref, wz_ref, bz_ref, out_ref):
    hi = hf_ref[0]
    ft = ft_ref[0]
    y2 = _pool2(_conv1(_pool3(_conv3(hi, wc1_ref, bc1_ref[...], D)),
                       wc2_ref, bc2_ref[...]))
    ci = jnp.concatenate([hi, ft], axis=1)
    z2 = _pool2(_conv1(_pool3(_conv3(ci, wcc1_ref, bcc1_ref[...], CD)),
                       wcc2_ref, bcc2_ref[...]))
    M = lax.dot_general(y2, z2, (((0,), (0,)), ((), ())),
                        preferred_element_type=jnp.float32)
    W = lax.dot_general(wy_ref[...], wz_ref[...], (((0,), (0,)), ((), ())),
                        preferred_element_type=jnp.float32)
    by = by_ref[0]
    bz = bz_ref[0]
    val = (jnp.sum(W * M)
           + by * jnp.sum(z2 * wz_ref[...])
           + bz * jnp.sum(y2 * wy_ref[...])
           + float(P2) * by * bz) / float(P2)
    out_ref[pl.program_id(0), :] = jnp.full((128,), jax.nn.sigmoid(val),
                                            jnp.float32)


def _readout(hfin, feat, w_c1, b_c1, w_c2, b_c2, w_cc1, b_cc1, w_cc2, b_cc2,
             W_y, b_y, W_z, b_z):
    full = lambda *shape: pl.BlockSpec(shape, lambda b: (0,) * len(shape))
    return pl.pallas_call(
        _readout_body,
        grid=(B,),
        in_specs=[
            pl.BlockSpec((1, L, D), lambda b: (b, 0, 0)),
            pl.BlockSpec((1, L, D), lambda b: (b, 0, 0)),
            full(3, D, D), full(D,), full(1, D, D), full(D,),
            full(3, CD, CD), full(CD,), full(1, CD, CD), full(CD,),
            full(1, D),
            pl.BlockSpec(memory_space=pltpu.SMEM),
            full(1, CD),
            pl.BlockSpec(memory_space=pltpu.SMEM),
        ],
        out_specs=pl.BlockSpec((B, 128), lambda b: (0, 0)),
        out_shape=jax.ShapeDtypeStruct((B, 128), jnp.float32),
    )(hfin, feat, w_c1, b_c1, w_c2, b_c2, w_cc1, b_cc1, w_cc2, b_cc2,
      W_y, b_y, W_z, b_z)


def kernel(h, edge_index, etype, W_msg, b_msg, W_ih, b_ih, W_hh, b_hh,
           w_c1, b_c1, w_c2, b_c2, w_cc1, b_cc1, w_cc2, b_cc2,
           W_y, b_y, W_z, b_z):
    src = edge_index[0]
    dst = edge_index[1]
    gidx = etype * NPAD + src
    npad_e = NS * EPW - E
    pad_ids = jnp.arange(npad_e, dtype=jnp.int32)
    gidx_t = jnp.concatenate(
        [gidx, (pad_ids * 97) % (T * NPAD)]).reshape(NS, NCH, CHUNK)
    gidx_all = jnp.stack([gidx_t + c * (T * NPAD) for c in range(NC)])
    dst_all = jnp.concatenate(
        [dst, N + pad_ids % (NPAD - N)]).reshape(NS, NCH, CHUNK)
    zinit = jnp.zeros((NPAD, DH), jnp.float32)
    hx = jnp.pad(h, ((0, NPAD - N), (0, 0)))

    trans = _trans(hx, W_msg, b_msg)
    for _ in range(STEPS):
        partials = _sc_gather_scatter(trans.reshape(NC * T * NPAD, DH),
                                      gidx_all, dst_all, zinit)
        hx, trans = _gru_trans(partials, hx, W_ih, b_ih, W_hh, b_hh,
                               W_msg, b_msg)

    hfin = hx[:N].reshape(B, L, D)
    feat = h.reshape(B, L, D)
    out = _readout(hfin, feat,
                   jnp.transpose(w_c1, (2, 0, 1)), b_c1,
                   jnp.transpose(w_c2, (2, 0, 1)), b_c2,
                   jnp.transpose(w_cc1, (2, 0, 1)), b_cc1,
                   jnp.transpose(w_cc2, (2, 0, 1)), b_cc2,
                   W_y, b_y, W_z, b_z)
    return out[:, 0]


# speedup vs baseline: 1.3965x; 1.1798x over previous
import functools

import jax
import jax.numpy as jnp
from jax import lax
from jax.experimental import pallas as pl
from jax.experimental.pallas import tpu as pltpu
from jax.experimental.pallas import tpu_sc as plsc

N = 10000
E = 320000
B = 8
L = N // B
D = 128
T = 4
STEPS = 6
CD = 2 * D

NPAD = 10240
NC = 2
NS = 16
DH = D // NC
CHUNK = 128
NBUF = 5
NCH = (-(-E // (NS * CHUNK)) + NBUF - 1) // NBUF * NBUF
EPW = NCH * CHUNK


def _trans_body(hx_ref, w_ref, b_ref, out_ref):
    x = hx_ref[...]
    w = w_ref[0]
    y = lax.dot_general(x, w, (((1,), (1,)), ((), ())),
                        preferred_element_type=jnp.float32)
    t = pl.program_id(0)
    y = (y + b_ref[t][None, :]).astype(jnp.bfloat16)
    out_ref[0, 0] = y[:, :DH]
    out_ref[1, 0] = y[:, DH:]


def _trans(hx, W_msg, b_msg):
    BN = 2048
    return pl.pallas_call(
        _trans_body,
        grid=(T, NPAD // BN),
        in_specs=[
            pl.BlockSpec((BN, D), lambda t, i: (i, 0)),
            pl.BlockSpec((1, D, D), lambda t, i: (t, 0, 0)),
            pl.BlockSpec((T, D), lambda t, i: (0, 0)),
        ],
        out_specs=pl.BlockSpec((NC, 1, BN, DH), lambda t, i: (0, t, i, 0)),
        out_shape=jax.ShapeDtypeStruct((NC, T, NPAD, DH), jnp.bfloat16),
    )(hx, W_msg, b_msg)


def _sc_gather_scatter(trans_flat, gidx_all, dst_all, zinit):
    mesh = plsc.VectorSubcoreMesh(core_axis_name="c", subcore_axis_name="s")

    @functools.partial(
        pl.kernel,
        mesh=mesh,
        compiler_params=pltpu.CompilerParams(use_tc_tiling_on_sc=False),
        out_type=jax.ShapeDtypeStruct((NC, NPAD, DH), jnp.bfloat16),
        scratch_types=[
            pltpu.VMEM((NCH, CHUNK), jnp.int32),
            pltpu.VMEM((NCH, CHUNK), jnp.int32),
            pltpu.VMEM_SHARED((NPAD, DH), jnp.bfloat16),
        ] + [pltpu.VMEM((CHUNK, DH), jnp.bfloat16) for _ in range(NBUF)]
          + [pltpu.SemaphoreType.DMA for _ in range(2 * NBUF)],
    )
    def run(trans_hbm, gidx_hbm, dst_hbm, zinit_hbm, out_hbm,
            gv, dv, acc, *bufs_sems):
        rbufs = bufs_sems[:NBUF]
        gsems = bufs_sems[NBUF:2 * NBUF]
        ssems = bufs_sems[2 * NBUF:]
        c = lax.axis_index("c")
        s = lax.axis_index("s")
        pltpu.sync_copy(gidx_hbm.at[c, s], gv)
        pltpu.sync_copy(dst_hbm.at[s], dv)
        rpt = NPAD // NS
        pltpu.sync_copy(zinit_hbm.at[pl.ds(s * rpt, rpt)],
                        acc.at[pl.ds(s * rpt, rpt)])
        plsc.subcore_barrier()

        def g_desc(b, j):
            return pltpu.make_async_copy(trans_hbm.at[gv.at[j]], rbufs[b],
                                         gsems[b])

        def s_desc(b, j):
            return pltpu.make_async_copy(rbufs[b], acc.at[dv.at[j]], ssems[b])

        for b in range(NBUF):
            g_desc(b, b).start()

        def body(jj, _):
            j0 = jj * NBUF
            for b in range(NBUF):
                g_desc(b, j0 + b).wait()
                s_desc(b, j0 + b).start(add=True)
            for b in range(NBUF):
                s_desc(b, j0 + b).wait()

                @pl.when(j0 + b + NBUF < NCH)
                def _():
                    g_desc(b, j0 + b + NBUF).start()
            return 0

        lax.fori_loop(0, NCH // NBUF, body, 0)
        plsc.subcore_barrier()
        pltpu.sync_copy(acc.at[pl.ds(s * rpt, rpt)],
                        out_hbm.at[c, pl.ds(s * rpt, rpt)])

    return run(trans_flat, gidx_all, dst_all, zinit)


def _gru_trans_body(p_ref, hx_ref, wih_ref, bih_ref, whh_ref, bhh_ref,
                    wmsg_ref, bmsg_ref, out_ref, tr_ref):
    a = jnp.concatenate([p_ref[0], p_ref[1]],
                        axis=1).astype(jnp.float32)
    x = hx_ref[...]
    gi = lax.dot_general(a, wih_ref[...], (((1,), (1,)), ((), ())),
                         preferred_element_type=jnp.float32) + bih_ref[...][None, :]
    gh = lax.dot_general(x, whh_ref[...], (((1,), (1,)), ((), ())),
                         preferred_element_type=jnp.float32) + bhh_ref[...][None, :]
    r = jax.nn.sigmoid(gi[:, :D] + gh[:, :D])
    z = jax.nn.sigmoid(gi[:, D:2 * D] + gh[:, D:2 * D])
    n = jnp.tanh(gi[:, 2 * D:] + r * gh[:, 2 * D:])
    hx_new = (1.0 - z) * n + z * x
    out_ref[...] = hx_new
    for t in range(T):
        y = lax.dot_general(hx_new, wmsg_ref[t], (((1,), (1,)), ((), ())),
                            preferred_element_type=jnp.float32)
        y = (y + bmsg_ref[t][None, :]).astype(jnp.bfloat16)
        tr_ref[0, t] = y[:, :DH]
        tr_ref[1, t] = y[:, DH:]


def _gru_trans(partials, hx, W_ih, b_ih, W_hh, b_hh, W_msg, b_msg):
    BG = 2048
    return pl.pallas_call(
        _gru_trans_body,
        grid=(NPAD // BG,),
        in_specs=[
            pl.BlockSpec((NC, BG, DH), lambda i: (0, i, 0)),
            pl.BlockSpec((BG, D), lambda i: (i, 0)),
            pl.BlockSpec((3 * D, D), lambda i: (0, 0)),
            pl.BlockSpec((3 * D,), lambda i: (0,)),
            pl.BlockSpec((3 * D, D), lambda i: (0, 0)),
            pl.BlockSpec((3 * D,), lambda i: (0,)),
            pl.BlockSpec((T, D, D), lambda i: (0, 0, 0)),
            pl.BlockSpec((T, D), lambda i: (0, 0)),
        ],
        out_specs=[
            pl.BlockSpec((BG, D), lambda i: (i, 0)),
            pl.BlockSpec((NC, T, BG, DH), lambda i: (0, 0, i, 0)),
        ],
        out_shape=[
            jax.ShapeDtypeStruct((NPAD, D), jnp.float32),
            jax.ShapeDtypeStruct((NC, T, NPAD, DH), jnp.bfloat16),
        ],
    )(partials, hx, W_ih, b_ih, W_hh, b_hh, W_msg, b_msg)


L1 = L - 2
P1 = (L1 - 3) // 2 + 1
P2 = (P1 - 2) // 2 + 1


def _conv3(x, w_ref, b, n_out):
    acc = None
    for k in range(3):
        xk = x[k:k + L1]
        yk = lax.dot_general(xk, w_ref[k], (((1,), (1,)), ((), ())),
                             preferred_element_type=jnp.float32)
        acc = yk if acc is None else acc + yk
    return jnp.maximum(acc + b[None, :], 0.0)


def _pool3(m3):
    c = m3.shape[1]
    r = m3.reshape(L1 // 2, 2, c)
    ev = r[:, 0, :]
    od = r[:, 1, :]
    pair = jnp.maximum(ev, od)
    return jnp.maximum(pair[0:P1], ev[1:P1 + 1])


def _conv1(x, w_ref, b):
    y = lax.dot_general(x, w_ref[0], (((1,), (1,)), ((), ())),
                        preferred_element_type=jnp.float32)
    return jnp.maximum(y + b[None, :], 0.0)


def _pool2(c2):
    c = c2.shape[1]
    r = c2[0:2 * P2].reshape(P2, 2, c)
    return jnp.maximum(r[:, 0, :], r[:, 1, :])


def _readout_body(hf_ref, ft_ref, wc1_ref, bc1_ref, wc2_ref, bc2_ref,
                  wcc1_ref, bcc1_ref, wcc2_ref, bcc2_ref,
                  wy_ref, by_ref, wz_ref, bz_ref, out_ref):
    hi = hf_ref[0]
    ft = ft_ref[0]
    y2 = _pool2(_conv1(_pool3(_conv3(hi, wc1_ref, bc1_ref[...], D)),
                       wc2_ref, bc2_ref[...]))
    ci = jnp.concatenate([hi, ft], axis=1)
    z2 = _pool2(_conv1(_pool3(_conv3(ci, wcc1_ref, bcc1_ref[...], CD)),
                       wcc2_ref, bcc2_ref[...]))
    M = lax.dot_general(y2, z2, (((0,), (0,)), ((), ())),
                        preferred_element_type=jnp.float32)
    W = lax.dot_general(wy_ref[...], wz_ref[...], (((0,), (0,)), ((), ())),
                        preferred_element_type=jnp.float32)
    by = by_ref[0]
    bz = bz_ref[0]
    val = (jnp.sum(W * M)
           + by * jnp.sum(z2 * wz_ref[...])
           + bz * jnp.sum(y2 * wy_ref[...])
           + float(P2) * by * bz) / float(P2)
    out_ref[pl.program_id(0), :] = jnp.full((128,), jax.nn.sigmoid(val),
                                            jnp.float32)


def _readout(hfin, feat, w_c1, b_c1, w_c2, b_c2, w_cc1, b_cc1, w_cc2, b_cc2,
             W_y, b_y, W_z, b_z):
    full = lambda *shape: pl.BlockSpec(shape, lambda b: (0,) * len(shape))
    return pl.pallas_call(
        _readout_body,
        grid=(B,),
        in_specs=[
            pl.BlockSpec((1, L, D), lambda b: (b, 0, 0)),
            pl.BlockSpec((1, L, D), lambda b: (b, 0, 0)),
            full(3, D, D), full(D,), full(1, D, D), full(D,),
            full(3, CD, CD), full(CD,), full(1, CD, CD), full(CD,),
            full(1, D),
            pl.BlockSpec(memory_space=pltpu.SMEM),
            full(1, CD),
            pl.BlockSpec(memory_space=pltpu.SMEM),
        ],
        out_specs=pl.BlockSpec((B, 128), lambda b: (0, 0)),
        out_shape=jax.ShapeDtypeStruct((B, 128), jnp.float32),
    )(hfin, feat, w_c1, b_c1, w_c2, b_c2, w_cc1, b_cc1, w_cc2, b_cc2,
      W_y, b_y, W_z, b_z)


def kernel(h, edge_index, etype, W_msg, b_msg, W_ih, b_ih, W_hh, b_hh,
           w_c1, b_c1, w_c2, b_c2, w_cc1, b_cc1, w_cc2, b_cc2,
           W_y, b_y, W_z, b_z):
    src = edge_index[0]
    dst = edge_index[1]
    gidx = etype * NPAD + src
    npad_e = NS * EPW - E
    pad_ids = jnp.arange(npad_e, dtype=jnp.int32)
    gidx_t = jnp.concatenate(
        [gidx, (pad_ids * 97) % (T * NPAD)]).reshape(NS, NCH, CHUNK)
    gidx_all = jnp.stack([gidx_t + c * (T * NPAD) for c in range(NC)])
    dst_all = jnp.concatenate(
        [dst, N + pad_ids % (NPAD - N)]).reshape(NS, NCH, CHUNK)
    zinit = jnp.zeros((NPAD, DH), jnp.bfloat16)
    hx = jnp.pad(h, ((0, NPAD - N), (0, 0)))

    trans = _trans(hx, W_msg, b_msg)
    for _ in range(STEPS):
        partials = _sc_gather_scatter(trans.reshape(NC * T * NPAD, DH),
                                      gidx_all, dst_all, zinit)
        hx, trans = _gru_trans(partials, hx, W_ih, b_ih, W_hh, b_hh,
                               W_msg, b_msg)

    hfin = hx[:N].reshape(B, L, D)
    feat = h.reshape(B, L, D)
    out = _readout(hfin, feat,
                   jnp.transpose(w_c1, (2, 0, 1)), b_c1,
                   jnp.transpose(w_c2, (2, 0, 1)), b_c2,
                   jnp.transpose(w_cc1, (2, 0, 1)), b_cc1,
                   jnp.transpose(w_cc2, (2, 0, 1)), b_cc2,
                   W_y, b_y, W_z, b_z)
    return out[:, 0]


# speedup vs baseline: 1.4312x; 1.0248x over previous
import functools

import jax
import jax.numpy as jnp
from jax import lax
from jax.experimental import pallas as pl
from jax.experimental.pallas import tpu as pltpu
from jax.experimental.pallas import tpu_sc as plsc

N = 10000
E = 320000
B = 8
L = N // B
D = 128
T = 4
STEPS = 6
CD = 2 * D

NPAD = 10240
NC = 2
NS = 16
DH = D // NC
CHUNK = 128
NBUF = 8
NCH = (-(-E // (NS * CHUNK)) + NBUF - 1) // NBUF * NBUF
EPW = NCH * CHUNK


def _trans_body(hx_ref, w_ref, b_ref, out_ref):
    x = hx_ref[...]
    w = w_ref[0]
    y = lax.dot_general(x, w, (((1,), (1,)), ((), ())),
                        preferred_element_type=jnp.float32)
    t = pl.program_id(0)
    y = (y + b_ref[t][None, :]).astype(jnp.bfloat16)
    out_ref[0, 0] = y[:, :DH]
    out_ref[1, 0] = y[:, DH:]


def _trans(hx, W_msg, b_msg):
    BN = 2048
    return pl.pallas_call(
        _trans_body,
        grid=(T, NPAD // BN),
        in_specs=[
            pl.BlockSpec((BN, D), lambda t, i: (i, 0)),
            pl.BlockSpec((1, D, D), lambda t, i: (t, 0, 0)),
            pl.BlockSpec((T, D), lambda t, i: (0, 0)),
        ],
        out_specs=pl.BlockSpec((NC, 1, BN, DH), lambda t, i: (0, t, i, 0)),
        out_shape=jax.ShapeDtypeStruct((NC, T, NPAD, DH), jnp.bfloat16),
    )(hx, W_msg, b_msg)


def _sc_gather_scatter(trans_flat, gidx_all, dst_all, zinit):
    mesh = plsc.VectorSubcoreMesh(core_axis_name="c", subcore_axis_name="s")

    @functools.partial(
        pl.kernel,
        mesh=mesh,
        compiler_params=pltpu.CompilerParams(use_tc_tiling_on_sc=False),
        out_type=jax.ShapeDtypeStruct((NC, NPAD, DH), jnp.bfloat16),
        scratch_types=[
            pltpu.VMEM((NCH, CHUNK), jnp.int32),
            pltpu.VMEM((NCH, CHUNK), jnp.int32),
            pltpu.VMEM_SHARED((NPAD, DH), jnp.bfloat16),
        ] + [pltpu.VMEM((CHUNK, DH), jnp.bfloat16) for _ in range(NBUF)]
          + [pltpu.SemaphoreType.DMA for _ in range(2 * NBUF)],
    )
    def run(trans_hbm, gidx_hbm, dst_hbm, zinit_hbm, out_hbm,
            gv, dv, acc, *bufs_sems):
        rbufs = bufs_sems[:NBUF]
        gsems = bufs_sems[NBUF:2 * NBUF]
        ssems = bufs_sems[2 * NBUF:]
        c = lax.axis_index("c")
        s = lax.axis_index("s")
        pltpu.sync_copy(gidx_hbm.at[c, s], gv)
        pltpu.sync_copy(dst_hbm.at[s], dv)
        rpt = NPAD // NS
        pltpu.sync_copy(zinit_hbm.at[pl.ds(s * rpt, rpt)],
                        acc.at[pl.ds(s * rpt, rpt)])
        plsc.subcore_barrier()

        def g_desc(b, j):
            return pltpu.make_async_copy(trans_hbm.at[gv.at[j]], rbufs[b],
                                         gsems[b])

        def s_desc(b, j):
            return pltpu.make_async_copy(rbufs[b], acc.at[dv.at[j]], ssems[b])

        for b in range(NBUF):
            g_desc(b, b).start()

        def body(jj, _):
            j0 = jj * NBUF
            for b in range(NBUF):
                g_desc(b, j0 + b).wait()
                s_desc(b, j0 + b).start(add=True)
            for b in range(NBUF):
                s_desc(b, j0 + b).wait()

                @pl.when(j0 + b + NBUF < NCH)
                def _():
                    g_desc(b, j0 + b + NBUF).start()
            return 0

        lax.fori_loop(0, NCH // NBUF, body, 0)
        plsc.subcore_barrier()
        pltpu.sync_copy(acc.at[pl.ds(s * rpt, rpt)],
                        out_hbm.at[c, pl.ds(s * rpt, rpt)])

    return run(trans_flat, gidx_all, dst_all, zinit)


def _gru_trans_body(p_ref, hx_ref, wih_ref, bih_ref, whh_ref, bhh_ref,
                    wmsg_ref, bmsg_ref, out_ref, tr_ref):
    a = jnp.concatenate([p_ref[0], p_ref[1]],
                        axis=1).astype(jnp.float32)
    x = hx_ref[...]
    gi = lax.dot_general(a, wih_ref[...], (((1,), (1,)), ((), ())),
                         preferred_element_type=jnp.float32) + bih_ref[...][None, :]
    gh = lax.dot_general(x, whh_ref[...], (((1,), (1,)), ((), ())),
                         preferred_element_type=jnp.float32) + bhh_ref[...][None, :]
    r = jax.nn.sigmoid(gi[:, :D] + gh[:, :D])
    z = jax.nn.sigmoid(gi[:, D:2 * D] + gh[:, D:2 * D])
    n = jnp.tanh(gi[:, 2 * D:] + r * gh[:, 2 * D:])
    hx_new = (1.0 - z) * n + z * x
    out_ref[...] = hx_new
    for t in range(T):
        y = lax.dot_general(hx_new, wmsg_ref[t], (((1,), (1,)), ((), ())),
                            preferred_element_type=jnp.float32)
        y = (y + bmsg_ref[t][None, :]).astype(jnp.bfloat16)
        tr_ref[0, t] = y[:, :DH]
        tr_ref[1, t] = y[:, DH:]


def _gru_trans(partials, hx, W_ih, b_ih, W_hh, b_hh, W_msg, b_msg):
    BG = 2048
    return pl.pallas_call(
        _gru_trans_body,
        grid=(NPAD // BG,),
        in_specs=[
            pl.BlockSpec((NC, BG, DH), lambda i: (0, i, 0)),
            pl.BlockSpec((BG, D), lambda i: (i, 0)),
            pl.BlockSpec((3 * D, D), lambda i: (0, 0)),
            pl.BlockSpec((3 * D,), lambda i: (0,)),
            pl.BlockSpec((3 * D, D), lambda i: (0, 0)),
            pl.BlockSpec((3 * D,), lambda i: (0,)),
            pl.BlockSpec((T, D, D), lambda i: (0, 0, 0)),
            pl.BlockSpec((T, D), lambda i: (0, 0)),
        ],
        out_specs=[
            pl.BlockSpec((BG, D), lambda i: (i, 0)),
            pl.BlockSpec((NC, T, BG, DH), lambda i: (0, 0, i, 0)),
        ],
        out_shape=[
            jax.ShapeDtypeStruct((NPAD, D), jnp.float32),
            jax.ShapeDtypeStruct((NC, T, NPAD, DH), jnp.bfloat16),
        ],
    )(partials, hx, W_ih, b_ih, W_hh, b_hh, W_msg, b_msg)


L1 = L - 2
P1 = (L1 - 3) // 2 + 1
P2 = (P1 - 2) // 2 + 1


def _conv3(x, w_ref, b, n_out):
    acc = None
    for k in range(3):
        xk = x[k:k + L1]
        yk = lax.dot_general(xk, w_ref[k], (((1,), (1,)), ((), ())),
                             preferred_element_type=jnp.float32)
        acc = yk if acc is None else acc + yk
    return jnp.maximum(acc + b[None, :], 0.0)


def _pool3(m3):
    c = m3.shape[1]
    r = m3.reshape(L1 // 2, 2, c)
    ev = r[:, 0, :]
    od = r[:, 1, :]
    pair = jnp.maximum(ev, od)
    return jnp.maximum(pair[0:P1], ev[1:P1 + 1])


def _conv1(x, w_ref, b):
    y = lax.dot_general(x, w_ref[0], (((1,), (1,)), ((), ())),
                        preferred_element_type=jnp.float32)
    return jnp.maximum(y + b[None, :], 0.0)


def _pool2(c2):
    c = c2.shape[1]
    r = c2[0:2 * P2].reshape(P2, 2, c)
    return jnp.maximum(r[:, 0, :], r[:, 1, :])


def _readout_body(hf_ref, ft_ref, wc1_ref, bc1_ref, wc2_ref, bc2_ref,
                  wcc1_ref, bcc1_ref, wcc2_ref, bcc2_ref,
                  wy_ref, by_ref, wz_ref, bz_ref, out_ref):
    hi = hf_ref[0]
    ft = ft_ref[0]
    y2 = _pool2(_conv1(_pool3(_conv3(hi, wc1_ref, bc1_ref[...], D)),
                       wc2_ref, bc2_ref[...]))
    ci = jnp.concatenate([hi, ft], axis=1)
    z2 = _pool2(_conv1(_pool3(_conv3(ci, wcc1_ref, bcc1_ref[...], CD)),
                       wcc2_ref, bcc2_ref[...]))
    M = lax.dot_general(y2, z2, (((0,), (0,)), ((), ())),
                        preferred_element_type=jnp.float32)
    W = lax.dot_general(wy_ref[...], wz_ref[...], (((0,), (0,)), ((), ())),
                        preferred_element_type=jnp.float32)
    by = by_ref[0]
    bz = bz_ref[0]
    val = (jnp.sum(W * M)
           + by * jnp.sum(z2 * wz_ref[...])
           + bz * jnp.sum(y2 * wy_ref[...])
           + float(P2) * by * bz) / float(P2)
    out_ref[pl.program_id(0), :] = jnp.full((128,), jax.nn.sigmoid(val),
                                            jnp.float32)


def _readout(hfin, feat, w_c1, b_c1, w_c2, b_c2, w_cc1, b_cc1, w_cc2, b_cc2,
             W_y, b_y, W_z, b_z):
    full = lambda *shape: pl.BlockSpec(shape, lambda b: (0,) * len(shape))
    return pl.pallas_call(
        _readout_body,
        grid=(B,),
        in_specs=[
            pl.BlockSpec((1, L, D), lambda b: (b, 0, 0)),
            pl.BlockSpec((1, L, D), lambda b: (b, 0, 0)),
            full(3, D, D), full(D,), full(1, D, D), full(D,),
            full(3, CD, CD), full(CD,), full(1, CD, CD), full(CD,),
            full(1, D),
            pl.BlockSpec(memory_space=pltpu.SMEM),
            full(1, CD),
            pl.BlockSpec(memory_space=pltpu.SMEM),
        ],
        out_specs=pl.BlockSpec((B, 128), lambda b: (0, 0)),
        out_shape=jax.ShapeDtypeStruct((B, 128), jnp.float32),
    )(hfin, feat, w_c1, b_c1, w_c2, b_c2, w_cc1, b_cc1, w_cc2, b_cc2,
      W_y, b_y, W_z, b_z)


def kernel(h, edge_index, etype, W_msg, b_msg, W_ih, b_ih, W_hh, b_hh,
           w_c1, b_c1, w_c2, b_c2, w_cc1, b_cc1, w_cc2, b_cc2,
           W_y, b_y, W_z, b_z):
    src = edge_index[0]
    dst = edge_index[1]
    gidx = etype * NPAD + src
    npad_e = NS * EPW - E
    pad_ids = jnp.arange(npad_e, dtype=jnp.int32)
    gidx_t = jnp.concatenate(
        [gidx, (pad_ids * 97) % (T * NPAD)]).reshape(NS, NCH, CHUNK)
    gidx_all = jnp.stack([gidx_t + c * (T * NPAD) for c in range(NC)])
    dst_all = jnp.concatenate(
        [dst, N + pad_ids % (NPAD - N)]).reshape(NS, NCH, CHUNK)
    zinit = jnp.zeros((NPAD, DH), jnp.bfloat16)
    hx = jnp.pad(h, ((0, NPAD - N), (0, 0)))

    trans = _trans(hx, W_msg, b_msg)
    for _ in range(STEPS):
        partials = _sc_gather_scatter(trans.reshape(NC * T * NPAD, DH),
                                      gidx_all, dst_all, zinit)
        hx, trans = _gru_trans(partials, hx, W_ih, b_ih, W_hh, b_hh,
                               W_msg, b_msg)

    hfin = hx[:N].reshape(B, L, D)
    feat = h.reshape(B, L, D)
    out = _readout(hfin, feat,
                   jnp.transpose(w_c1, (2, 0, 1)), b_c1,
                   jnp.transpose(w_c2, (2, 0, 1)), b_c2,
                   jnp.transpose(w_cc1, (2, 0, 1)), b_cc1,
                   jnp.transpose(w_cc2, (2, 0, 1)), b_cc2,
                   W_y, b_y, W_z, b_z)
    return out[:, 0]


# speedup vs baseline: 1.4345x; 1.0023x over previous
import functools

import jax
import jax.numpy as jnp
from jax import lax
from jax.experimental import pallas as pl
from jax.experimental.pallas import tpu as pltpu
from jax.experimental.pallas import tpu_sc as plsc

N = 10000
E = 320000
B = 8
L = N // B
D = 128
T = 4
STEPS = 6
CD = 2 * D

NPAD = 10240
NC = 2
NS = 16
DH = D // NC
CHUNK = 128
NBUF = 8
NCH = (-(-E // (NS * CHUNK)) + NBUF - 1) // NBUF * NBUF
EPW = NCH * CHUNK


def _trans_body(hx_ref, w_ref, b_ref, out_ref):
    x = hx_ref[...].astype(jnp.bfloat16)
    w = w_ref[0].astype(jnp.bfloat16)
    y = lax.dot_general(x, w, (((1,), (1,)), ((), ())),
                        preferred_element_type=jnp.float32)
    t = pl.program_id(0)
    y = (y + b_ref[t][None, :]).astype(jnp.bfloat16)
    out_ref[0, 0] = y[:, :DH]
    out_ref[1, 0] = y[:, DH:]


def _trans(hx, W_msg, b_msg):
    BN = 2048
    return pl.pallas_call(
        _trans_body,
        grid=(T, NPAD // BN),
        in_specs=[
            pl.BlockSpec((BN, D), lambda t, i: (i, 0)),
            pl.BlockSpec((1, D, D), lambda t, i: (t, 0, 0)),
            pl.BlockSpec((T, D), lambda t, i: (0, 0)),
        ],
        out_specs=pl.BlockSpec((NC, 1, BN, DH), lambda t, i: (0, t, i, 0)),
        out_shape=jax.ShapeDtypeStruct((NC, T, NPAD, DH), jnp.bfloat16),
    )(hx, W_msg, b_msg)


def _sc_gather_scatter(trans_flat, gidx_all, dst_all, zinit):
    mesh = plsc.VectorSubcoreMesh(core_axis_name="c", subcore_axis_name="s")

    @functools.partial(
        pl.kernel,
        mesh=mesh,
        compiler_params=pltpu.CompilerParams(use_tc_tiling_on_sc=False),
        out_type=jax.ShapeDtypeStruct((NC, NPAD, DH), jnp.bfloat16),
        scratch_types=[
            pltpu.VMEM((NCH, CHUNK), jnp.int32),
            pltpu.VMEM((NCH, CHUNK), jnp.int32),
            pltpu.VMEM_SHARED((NPAD, DH), jnp.bfloat16),
        ] + [pltpu.VMEM((CHUNK, DH), jnp.bfloat16) for _ in range(NBUF)]
          + [pltpu.SemaphoreType.DMA for _ in range(2 * NBUF)],
    )
    def run(trans_hbm, gidx_hbm, dst_hbm, zinit_hbm, out_hbm,
            gv, dv, acc, *bufs_sems):
        rbufs = bufs_sems[:NBUF]
        gsems = bufs_sems[NBUF:2 * NBUF]
        ssems = bufs_sems[2 * NBUF:]
        c = lax.axis_index("c")
        s = lax.axis_index("s")
        pltpu.sync_copy(gidx_hbm.at[c, s], gv)
        pltpu.sync_copy(dst_hbm.at[s], dv)
        rpt = NPAD // NS
        pltpu.sync_copy(zinit_hbm.at[pl.ds(s * rpt, rpt)],
                        acc.at[pl.ds(s * rpt, rpt)])
        plsc.subcore_barrier()

        def g_desc(b, j):
            return pltpu.make_async_copy(trans_hbm.at[gv.at[j]], rbufs[b],
                                         gsems[b])

        def s_desc(b, j):
            return pltpu.make_async_copy(rbufs[b], acc.at[dv.at[j]], ssems[b])

        for b in range(NBUF):
            g_desc(b, b).start()

        def body(jj, _):
            j0 = jj * NBUF
            for b in range(NBUF):
                g_desc(b, j0 + b).wait()
                s_desc(b, j0 + b).start(add=True)
            for b in range(NBUF):
                s_desc(b, j0 + b).wait()

                @pl.when(j0 + b + NBUF < NCH)
                def _():
                    g_desc(b, j0 + b + NBUF).start()
            return 0

        lax.fori_loop(0, NCH // NBUF, body, 0)
        plsc.subcore_barrier()
        pltpu.sync_copy(acc.at[pl.ds(s * rpt, rpt)],
                        out_hbm.at[c, pl.ds(s * rpt, rpt)])

    return run(trans_flat, gidx_all, dst_all, zinit)


def _gru_trans_body(p_ref, hx_ref, wih_ref, bih_ref, whh_ref, bhh_ref,
                    wmsg_ref, bmsg_ref, out_ref, tr_ref):
    a = jnp.concatenate([p_ref[0], p_ref[1]], axis=1)
    x = hx_ref[...]
    xb = x.astype(jnp.bfloat16)
    gi = lax.dot_general(a, wih_ref[...].astype(jnp.bfloat16),
                         (((1,), (1,)), ((), ())),
                         preferred_element_type=jnp.float32) + bih_ref[...][None, :]
    gh = lax.dot_general(xb, whh_ref[...].astype(jnp.bfloat16),
                         (((1,), (1,)), ((), ())),
                         preferred_element_type=jnp.float32) + bhh_ref[...][None, :]
    r = jax.nn.sigmoid(gi[:, :D] + gh[:, :D])
    z = jax.nn.sigmoid(gi[:, D:2 * D] + gh[:, D:2 * D])
    n = jnp.tanh(gi[:, 2 * D:] + r * gh[:, 2 * D:])
    hx_new = (1.0 - z) * n + z * x
    out_ref[...] = hx_new
    hb = hx_new.astype(jnp.bfloat16)
    for t in range(T):
        y = lax.dot_general(hb, wmsg_ref[t].astype(jnp.bfloat16),
                            (((1,), (1,)), ((), ())),
                            preferred_element_type=jnp.float32)
        y = (y + bmsg_ref[t][None, :]).astype(jnp.bfloat16)
        tr_ref[0, t] = y[:, :DH]
        tr_ref[1, t] = y[:, DH:]


def _gru_last_body(p_ref, hx_ref, wih_ref, bih_ref, whh_ref, bhh_ref,
                   out_ref):
    a = jnp.concatenate([p_ref[0], p_ref[1]], axis=1)
    x = hx_ref[...]
    xb = x.astype(jnp.bfloat16)
    gi = lax.dot_general(a, wih_ref[...].astype(jnp.bfloat16),
                         (((1,), (1,)), ((), ())),
                         preferred_element_type=jnp.float32) + bih_ref[...][None, :]
    gh = lax.dot_general(xb, whh_ref[...].astype(jnp.bfloat16),
                         (((1,), (1,)), ((), ())),
                         preferred_element_type=jnp.float32) + bhh_ref[...][None, :]
    r = jax.nn.sigmoid(gi[:, :D] + gh[:, :D])
    z = jax.nn.sigmoid(gi[:, D:2 * D] + gh[:, D:2 * D])
    n = jnp.tanh(gi[:, 2 * D:] + r * gh[:, 2 * D:])
    out_ref[...] = (1.0 - z) * n + z * x


def _gru_last(partials, hx, W_ih, b_ih, W_hh, b_hh):
    BG = 2048
    return pl.pallas_call(
        _gru_last_body,
        grid=(NPAD // BG,),
        in_specs=[
            pl.BlockSpec((NC, BG, DH), lambda i: (0, i, 0)),
            pl.BlockSpec((BG, D), lambda i: (i, 0)),
            pl.BlockSpec((3 * D, D), lambda i: (0, 0)),
            pl.BlockSpec((3 * D,), lambda i: (0,)),
            pl.BlockSpec((3 * D, D), lambda i: (0, 0)),
            pl.BlockSpec((3 * D,), lambda i: (0,)),
        ],
        out_specs=pl.BlockSpec((BG, D), lambda i: (i, 0)),
        out_shape=jax.ShapeDtypeStruct((NPAD, D), jnp.float32),
    )(partials, hx, W_ih, b_ih, W_hh, b_hh)


def _gru_trans(partials, hx, W_ih, b_ih, W_hh, b_hh, W_msg, b_msg):
    BG = 2048
    return pl.pallas_call(
        _gru_trans_body,
        grid=(NPAD // BG,),
        in_specs=[
            pl.BlockSpec((NC, BG, DH), lambda i: (0, i, 0)),
            pl.BlockSpec((BG, D), lambda i: (i, 0)),
            pl.BlockSpec((3 * D, D), lambda i: (0, 0)),
            pl.BlockSpec((3 * D,), lambda i: (0,)),
            pl.BlockSpec((3 * D, D), lambda i: (0, 0)),
            pl.BlockSpec((3 * D,), lambda i: (0,)),
            pl.BlockSpec((T, D, D), lambda i: (0, 0, 0)),
            pl.BlockSpec((T, D), lambda i: (0, 0)),
        ],
        out_specs=[
            pl.BlockSpec((BG, D), lambda i: (i, 0)),
            pl.BlockSpec((NC, T, BG, DH), lambda i: (0, 0, i, 0)),
        ],
        out_shape=[
            jax.ShapeDtypeStruct((NPAD, D), jnp.float32),
            jax.ShapeDtypeStruct((NC, T, NPAD, DH), jnp.bfloat16),
        ],
    )(partials, hx, W_ih, b_ih, W_hh, b_hh, W_msg, b_msg)


L1 = L - 2
P1 = (L1 - 3) // 2 + 1
P2 = (P1 - 2) // 2 + 1


def _conv3(x, w_ref, b, n_out):
    xb = x.astype(jnp.bfloat16)
    acc = None
    for k in range(3):
        xk = xb[k:k + L1]
        yk = lax.dot_general(xk, w_ref[k].astype(jnp.bfloat16),
                             (((1,), (1,)), ((), ())),
                             preferred_element_type=jnp.float32)
        acc = yk if acc is None else acc + yk
    return jnp.maximum(acc + b[None, :], 0.0)


def _pool3(m3):
    c = m3.shape[1]
    r = m3.reshape(L1 // 2, 2, c)
    ev = r[:, 0, :]
    od = r[:, 1, :]
    pair = jnp.maximum(ev, od)
    return jnp.maximum(pair[0:P1], ev[1:P1 + 1])


def _conv1(x, w_ref, b):
    y = lax.dot_general(x.astype(jnp.bfloat16),
                        w_ref[0].astype(jnp.bfloat16),
                        (((1,), (1,)), ((), ())),
                        preferred_element_type=jnp.float32)
    return jnp.maximum(y + b[None, :], 0.0)


def _pool2(c2):
    c = c2.shape[1]
    r = c2[0:2 * P2].reshape(P2, 2, c)
    return jnp.maximum(r[:, 0, :], r[:, 1, :])


def _readout_body(hf_ref, ft_ref, wc1_ref, bc1_ref, wc2_ref, bc2_ref,
                  wcc1_ref, bcc1_ref, wcc2_ref, bcc2_ref,
                  wy_ref, by_ref, wz_ref, bz_ref, out_ref):
    hi = hf_ref[0]
    ft = ft_ref[0]
    y2 = _pool2(_conv1(_pool3(_conv3(hi, wc1_ref, bc1_ref[...], D)),
                       wc2_ref, bc2_ref[...]))
    ci = jnp.concatenate([hi, ft], axis=1)
    z2 = _pool2(_conv1(_pool3(_conv3(ci, wcc1_ref, bcc1_ref[...], CD)),
                       wcc2_ref, bcc2_ref[...]))
    M = lax.dot_general(y2, z2, (((0,), (0,)), ((), ())),
                        preferred_element_type=jnp.float32)
    W = lax.dot_general(wy_ref[...], wz_ref[...], (((0,), (0,)), ((), ())),
                        preferred_element_type=jnp.float32)
    by = by_ref[0]
    bz = bz_ref[0]
    val = (jnp.sum(W * M)
           + by * jnp.sum(z2 * wz_ref[...])
           + bz * jnp.sum(y2 * wy_ref[...])
           + float(P2) * by * bz) / float(P2)
    out_ref[pl.program_id(0), :] = jnp.full((128,), jax.nn.sigmoid(val),
                                            jnp.float32)


def _readout(hfin, feat, w_c1, b_c1, w_c2, b_c2, w_cc1, b_cc1, w_cc2, b_cc2,
             W_y, b_y, W_z, b_z):
    full = lambda *shape: pl.BlockSpec(shape, lambda b: (0,) * len(shape))
    return pl.pallas_call(
        _readout_body,
        grid=(B,),
        in_specs=[
            pl.BlockSpec((1, L, D), lambda b: (b, 0, 0)),
            pl.BlockSpec((1, L, D), lambda b: (b, 0, 0)),
            full(3, D, D), full(D,), full(1, D, D), full(D,),
            full(3, CD, CD), full(CD,), full(1, CD, CD), full(CD,),
            full(1, D),
            pl.BlockSpec(memory_space=pltpu.SMEM),
            full(1, CD),
            pl.BlockSpec(memory_space=pltpu.SMEM),
        ],
        out_specs=pl.BlockSpec((B, 128), lambda b: (0, 0)),
        out_shape=jax.ShapeDtypeStruct((B, 128), jnp.float32),
    )(hfin, feat, w_c1, b_c1, w_c2, b_c2, w_cc1, b_cc1, w_cc2, b_cc2,
      W_y, b_y, W_z, b_z)


def kernel(h, edge_index, etype, W_msg, b_msg, W_ih, b_ih, W_hh, b_hh,
           w_c1, b_c1, w_c2, b_c2, w_cc1, b_cc1, w_cc2, b_cc2,
           W_y, b_y, W_z, b_z):
    src = edge_index[0]
    dst = edge_index[1]
    gidx = etype * NPAD + src
    npad_e = NS * EPW - E
    pad_ids = jnp.arange(npad_e, dtype=jnp.int32)
    gidx_t = jnp.concatenate(
        [gidx, (pad_ids * 97) % (T * NPAD)]).reshape(NS, NCH, CHUNK)
    gidx_all = jnp.stack([gidx_t + c * (T * NPAD) for c in range(NC)])
    dst_all = jnp.concatenate(
        [dst, N + pad_ids % (NPAD - N)]).reshape(NS, NCH, CHUNK)
    zinit = jnp.zeros((NPAD, DH), jnp.bfloat16)
    hx = jnp.pad(h, ((0, NPAD - N), (0, 0)))

    trans = _trans(hx, W_msg, b_msg)
    for step in range(STEPS):
        partials = _sc_gather_scatter(trans.reshape(NC * T * NPAD, DH),
                                      gidx_all, dst_all, zinit)
        if step + 1 < STEPS:
            hx, trans = _gru_trans(partials, hx, W_ih, b_ih, W_hh, b_hh,
                                   W_msg, b_msg)
        else:
            hx = _gru_last(partials, hx, W_ih, b_ih, W_hh, b_hh)

    hfin = hx[:N].reshape(B, L, D)
    feat = h.reshape(B, L, D)
    out = _readout(hfin, feat,
                   jnp.transpose(w_c1, (2, 0, 1)), b_c1,
                   jnp.transpose(w_c2, (2, 0, 1)), b_c2,
                   jnp.transpose(w_cc1, (2, 0, 1)), b_cc1,
                   jnp.transpose(w_cc2, (2, 0, 1)), b_cc2,
                   W_y, b_y, W_z, b_z)
    return out[:, 0]
